# baseline (speedup 1.0000x reference)
import functools
import math

import jax
import jax.numpy as jnp
import numpy as np
from jax import lax
from jax.experimental import pallas as pl
from jax.experimental.pallas import tpu as pltpu
from jax.experimental.pallas import tpu_sc as plsc

BLK = 128
NB = 16
S = NB * BLK
D = 64
NSPECIAL = 4
CAT = S + S // 2 + S // 4 + S // 8 + BLK
MASK_RATIOS = ((1, 0.0, 0.05), (2, 0.05, 0.15), (4, 0.15, 0.25),
               (8, 0.25, 0.5), (0, 0.5, 1.0))
SIM_T2, SIM_T4, SIM_T8 = 0.75, 0.7, 0.7
NEG = np.float32(-np.inf)
HI = lax.Precision.HIGHEST


def _make_adj_table():
    rows = np.arange(BLK)[:, None]
    cols = np.arange(BLK)[None, :]
    table = np.zeros((4, BLK, BLK), np.float32)
    for n, p in enumerate((1, 2, 4, 8)):
        c = np.clip(rows + 1 - cols * p, 0, p).astype(np.float64)
        with np.errstate(divide="ignore"):
            table[n] = np.where(c > 0, np.log(c) - math.log(p),
                                -np.inf).astype(np.float32)
    return table


def _make_band_table():
    t = np.zeros((NB, NB), np.int32)
    for r in range(NB):
        valid = r + 1
        for value, sr, er in MASK_RATIOS:
            lo = min(int(valid * sr), NB)
            hi = min(int(valid * er), NB)
            t[r, lo:hi] = value
    return t


def _make_forced_table():
    t = np.full((NB, NB), -1, np.int32)
    for r in range(NB):
        for c in range(NB):
            if c > r:
                t[r, c] = 0
            elif c >= NB - NSPECIAL or r >= NB - NSPECIAL or c == r or c == 0:
                t[r, c] = 1
    return t


_ADJ_TABLE = _make_adj_table()
_BAND_TABLE = _make_band_table()
_FORCED_TABLE = _make_forced_table()


def _pair_cos(a, b):
    num = (a * b).sum(-1)
    den = jnp.sqrt((a * a).sum(-1)) * jnp.sqrt((b * b).sum(-1)) + 1e-6
    return (num / den).mean(-1)


def _est_body(q_ref, k_ref, attn_ref, val_ref):
    k = k_ref[0]
    q = q_ref[0]

    qs = q.reshape(NB, 8, 16, D)[:, :, 0, :].mean(axis=1)
    ks = k.reshape(NB, 8, 16, D)[:, :, 0, :].mean(axis=1)
    scores = jnp.dot(qs, ks.T, precision=HI) * (1.0 / math.sqrt(D))
    row = lax.broadcasted_iota(jnp.int32, (NB, NB), 0)
    col = lax.broadcasted_iota(jnp.int32, (NB, NB), 1)
    scores = jnp.where(col <= row, scores, NEG)
    mx = jnp.max(scores, axis=-1, keepdims=True)
    e = jnp.exp(scores - mx)
    attn_ref[0] = e / jnp.sum(e, axis=-1, keepdims=True)

    p2 = k.reshape(NB, BLK // 2, 2, D)
    sim2 = _pair_cos(p2[:, :, 0, :], p2[:, :, 1, :])
    kk2 = p2.mean(axis=2)
    p4 = kk2.reshape(NB, BLK // 4, 2, D)
    sim4 = _pair_cos(p4[:, :, 0, :], p4[:, :, 1, :])
    kk4 = p4.mean(axis=2)
    p8 = kk4.reshape(NB, BLK // 8, 2, D)
    sim8 = _pair_cos(p8[:, :, 0, :], p8[:, :, 1, :])
    val = jnp.where(sim2 >= SIM_T2,
                    jnp.where(sim4 >= SIM_T4,
                              jnp.where(sim8 >= SIM_T8, 8, 4), 2), 1)
    val_ref[0, 0] = val.astype(jnp.int32)


def _pool_body(k_ref, v_ref, kcat_ref, vcat_ref):
    k = k_ref[0]
    v = v_ref[0]
    k2 = k.reshape(S // 2, 2, D).mean(axis=1)
    k4 = k2.reshape(S // 4, 2, D).mean(axis=1)
    k8 = k4.reshape(S // 8, 2, D).mean(axis=1)
    kcat_ref[0] = jnp.concatenate(
        [k, k2, k4, k8, jnp.zeros((BLK, D), jnp.float32)], axis=0)
    v2 = v.reshape(S // 2, 2, D).mean(axis=1)
    v4 = v2.reshape(S // 4, 2, D).mean(axis=1)
    v8 = v4.reshape(S // 8, 2, D).mean(axis=1)
    vcat_ref[0] = jnp.concatenate(
        [v, v2, v4, v8, jnp.zeros((BLK, D), jnp.float32)], axis=0)


def _mask_sc_body(attn_hbm, val_hbm, band_hbm, forced_hbm, out_hbm,
                  band_v, forced_v, arow_v, ranks_v, valrow_v, outrow_v):
    nrows = attn_hbm.shape[0]
    wid = lax.axis_index("s") * 2 + lax.axis_index("c")
    per_w = nrows // 32
    base = wid * per_w

    pltpu.sync_copy(band_hbm, band_v)
    pltpu.sync_copy(forced_hbm, forced_v)

    idx = lax.iota(jnp.int32, NB)
    for t in range(per_w):
        row = base + t
        h = lax.shift_right_logical(row, 4)
        r = lax.bitwise_and(row, 15)
        pltpu.sync_copy(attn_hbm.at[row], arow_v)
        pltpu.sync_copy(val_hbm.at[h], valrow_v)
        a = arow_v[...]
        _, order = plsc.sort_key_val(a, idx, descending=True)
        plsc.store_scatter(ranks_v, [order], idx)
        ranks = ranks_v[...]
        band = plsc.load_gather(band_v, [jnp.full((NB,), r, jnp.int32),
                                         ranks])
        forced = forced_v[r]
        m = jnp.where(forced >= 0, forced, band)
        outrow_v[...] = jnp.minimum(m, valrow_v[...])
        pltpu.sync_copy(outrow_v, out_hbm.at[row])


def _mask_sc(attn2, val2):
    nrows = attn2.shape[0]
    kern = functools.partial(
        pl.kernel,
        mesh=plsc.VectorSubcoreMesh(core_axis_name="c", subcore_axis_name="s"),
        out_type=jax.ShapeDtypeStruct((nrows, NB), jnp.int32),
        compiler_params=pltpu.CompilerParams(needs_layout_passes=False),
        scratch_types=[
            pltpu.VMEM((NB, NB), jnp.int32),
            pltpu.VMEM((NB, NB), jnp.int32),
            pltpu.VMEM((NB,), jnp.float32),
            pltpu.VMEM((NB,), jnp.int32),
            pltpu.VMEM((NB,), jnp.int32),
            pltpu.VMEM((NB,), jnp.int32),
        ],
    )(_mask_sc_body)
    return kern(attn2, val2, jnp.asarray(_BAND_TABLE),
                jnp.asarray(_FORCED_TABLE))


def _attn_body(mask_ref, adj_ref, q_ref, kcat_ref, vcat_ref, o_ref):
    i = pl.program_id(1)
    scale = np.float32(1.0 / math.sqrt(D))
    qs = q_ref[0] * scale
    coli = lax.broadcasted_iota(jnp.int32, (BLK, BLK), 1)

    pd = mask_ref[0, i, i]
    idx = ((pd > 1).astype(jnp.int32) + (pd > 2).astype(jnp.int32)
           + (pd > 4).astype(jnp.int32))
    adj = adj_ref[idx]

    def logits(j, p):
        pe = jnp.maximum(p, 1)
        w = 128 // pe
        start = (4096 - 8192 // (2 * pe)) + j * w
        kblk = kcat_ref[0, pl.ds(start, BLK), :]
        vblk = vcat_ref[0, pl.ds(start, BLK), :]
        sj = jnp.dot(qs, kblk.T, precision=HI)
        offdiag = jnp.where((coli < w) & (p > 0), 0.0, NEG)
        sj = sj + jnp.where(j == i, adj, offdiag)
        return sj, vblk

    UNROLL = 8

    def body(t, carry):
        m, l, acc = carry
        js = [UNROLL * t + u for u in range(UNROLL)]
        ps = [mask_ref[0, i, js[0]]]
        ps += [jnp.where(js[u] <= i,
                         mask_ref[0, i, jnp.minimum(js[u], NB - 1)], 0)
               for u in range(1, UNROLL)]
        sv = [logits(js[u], ps[u]) for u in range(UNROLL)]
        mx = jnp.max(sv[0][0], axis=1, keepdims=True)
        for u in range(1, UNROLL):
            mx = jnp.maximum(mx, jnp.max(sv[u][0], axis=1, keepdims=True))
        m_new = jnp.maximum(m, mx)
        alpha = jnp.exp(m - m_new)
        es = [jnp.exp(sj - m_new) for sj, _ in sv]
        l_new = l * alpha
        for e in es:
            l_new = l_new + jnp.sum(e, axis=1, keepdims=True)
        acc_new = acc * alpha
        for e, (_, vb) in zip(es, sv):
            acc_new = acc_new + jnp.dot(e, vb)
        return m_new, l_new, acc_new

    m0 = jnp.full((BLK, 1), NEG, jnp.float32)
    l0 = jnp.zeros((BLK, 1), jnp.float32)
    a0 = jnp.zeros((BLK, D), jnp.float32)
    m, l, acc = lax.fori_loop(0, (i + UNROLL) // UNROLL, body, (m0, l0, a0))
    o_ref[0] = acc / l


def _estimate(q3, k3, interpret=False):
    H = q3.shape[0]
    return pl.pallas_call(
        _est_body,
        grid=(H,),
        in_specs=[pl.BlockSpec((1, S, D), lambda h: (h, 0, 0))] * 2,
        out_specs=[
            pl.BlockSpec((1, NB, NB), lambda h: (h, 0, 0)),
            pl.BlockSpec((1, 1, NB), lambda h: (h, 0, 0)),
        ],
        out_shape=[
            jax.ShapeDtypeStruct((H, NB, NB), jnp.float32),
            jax.ShapeDtypeStruct((H, 1, NB), jnp.int32),
        ],
        interpret=interpret,
    )(q3, k3)


def _pool(k3, v3, interpret=False):
    H = k3.shape[0]
    return pl.pallas_call(
        _pool_body,
        grid=(H,),
        in_specs=[pl.BlockSpec((1, S, D), lambda h: (h, 0, 0))] * 2,
        out_specs=[
            pl.BlockSpec((1, CAT, D), lambda h: (h, 0, 0)),
            pl.BlockSpec((1, CAT, D), lambda h: (h, 0, 0)),
        ],
        out_shape=[
            jax.ShapeDtypeStruct((H, CAT, D), jnp.float32),
            jax.ShapeDtypeStruct((H, CAT, D), jnp.float32),
        ],
        interpret=interpret,
    )(k3, v3)


def _attend(mask, q3, kcat, vcat, interpret=False):
    H = q3.shape[0]
    return pl.pallas_call(
        _attn_body,
        grid=(H, NB),
        in_specs=[
            pl.BlockSpec((1, NB, NB), lambda h, i: (h, 0, 0),
                         memory_space=pltpu.SMEM),
            pl.BlockSpec((4, BLK, BLK), lambda h, i: (0, 0, 0)),
            pl.BlockSpec((1, BLK, D), lambda h, i: (h, i, 0)),
            pl.BlockSpec((1, CAT, D), lambda h, i: (h, 0, 0)),
            pl.BlockSpec((1, CAT, D), lambda h, i: (h, 0, 0)),
        ],
        out_specs=pl.BlockSpec((1, BLK, D), lambda h, i: (h, i, 0)),
        out_shape=jax.ShapeDtypeStruct((H, S, D), jnp.float32),
        interpret=interpret,
    )(mask, _ADJ_TABLE, q3, kcat, vcat)


def kernel(q, k, v):
    B, H, s, d = q.shape
    assert s == S and d == D
    q3 = q.reshape(B * H, S, D)
    k3 = k.reshape(B * H, S, D)
    v3 = v.reshape(B * H, S, D)
    attn, val = _estimate(q3, k3)
    kcat, vcat = _pool(k3, v3)
    mask2 = _mask_sc(attn.reshape(B * H * NB, NB), val.reshape(B * H, NB))
    out = _attend(mask2.reshape(B * H, NB, NB), q3, kcat, vcat)
    return out.reshape(B, H, S, D)


# speedup vs baseline: 1.0518x; 1.0518x over previous
import functools
import math

import jax
import jax.numpy as jnp
import numpy as np
from jax import lax
from jax.experimental import pallas as pl
from jax.experimental.pallas import tpu as pltpu
from jax.experimental.pallas import tpu_sc as plsc

BLK = 128
NB = 16
S = NB * BLK
D = 64
NSPECIAL = 4
CAT = S + S // 2 + S // 4 + S // 8 + BLK
MASK_RATIOS = ((1, 0.0, 0.05), (2, 0.05, 0.15), (4, 0.15, 0.25),
               (8, 0.25, 0.5), (0, 0.5, 1.0))
SIM_T2, SIM_T4, SIM_T8 = 0.75, 0.7, 0.7
NEG = np.float32(-np.inf)
HI = lax.Precision.HIGHEST


def _make_adj_table():
    rows = np.arange(BLK)[:, None]
    cols = np.arange(BLK)[None, :]
    table = np.zeros((4, BLK, BLK), np.float32)
    for n, p in enumerate((1, 2, 4, 8)):
        c = np.clip(rows + 1 - cols * p, 0, p).astype(np.float64)
        with np.errstate(divide="ignore"):
            table[n] = np.where(c > 0, np.log(c) - math.log(p),
                                -np.inf).astype(np.float32)
    return table


def _make_band_table():
    t = np.zeros((NB, NB), np.int32)
    for r in range(NB):
        valid = r + 1
        for value, sr, er in MASK_RATIOS:
            lo = min(int(valid * sr), NB)
            hi = min(int(valid * er), NB)
            t[r, lo:hi] = value
    return t


def _make_forced_table():
    t = np.full((NB, NB), -1, np.int32)
    for r in range(NB):
        for c in range(NB):
            if c > r:
                t[r, c] = 0
            elif c >= NB - NSPECIAL or r >= NB - NSPECIAL or c == r or c == 0:
                t[r, c] = 1
    return t


_ADJ_TABLE = _make_adj_table()
_BAND_TABLE = _make_band_table()
_FORCED_TABLE = _make_forced_table()


def _pair_cos(a, b):
    num = (a * b).sum(-1)
    den = jnp.sqrt((a * a).sum(-1)) * jnp.sqrt((b * b).sum(-1)) + 1e-6
    return (num / den).mean(-1)


def _est_body(q_ref, k_ref, attn_ref, val_ref):
    k = k_ref[0]
    q = q_ref[0]

    qs = q.reshape(NB, 8, 16, D)[:, :, 0, :].mean(axis=1)
    ks = k.reshape(NB, 8, 16, D)[:, :, 0, :].mean(axis=1)
    scores = jnp.dot(qs, ks.T, precision=HI) * (1.0 / math.sqrt(D))
    row = lax.broadcasted_iota(jnp.int32, (NB, NB), 0)
    col = lax.broadcasted_iota(jnp.int32, (NB, NB), 1)
    scores = jnp.where(col <= row, scores, NEG)
    mx = jnp.max(scores, axis=-1, keepdims=True)
    e = jnp.exp(scores - mx)
    attn_ref[0] = e / jnp.sum(e, axis=-1, keepdims=True)

    p2 = k.reshape(NB, BLK // 2, 2, D)
    sim2 = _pair_cos(p2[:, :, 0, :], p2[:, :, 1, :])
    kk2 = p2.mean(axis=2)
    p4 = kk2.reshape(NB, BLK // 4, 2, D)
    sim4 = _pair_cos(p4[:, :, 0, :], p4[:, :, 1, :])
    kk4 = p4.mean(axis=2)
    p8 = kk4.reshape(NB, BLK // 8, 2, D)
    sim8 = _pair_cos(p8[:, :, 0, :], p8[:, :, 1, :])
    val = jnp.where(sim2 >= SIM_T2,
                    jnp.where(sim4 >= SIM_T4,
                              jnp.where(sim8 >= SIM_T8, 8, 4), 2), 1)
    val_ref[0, 0] = val.astype(jnp.int32)


def _pool_body(k_ref, v_ref, kcat_ref, vcat_ref):
    k = k_ref[0]
    v = v_ref[0]
    k2 = k.reshape(S // 2, 2, D).mean(axis=1)
    k4 = k2.reshape(S // 4, 2, D).mean(axis=1)
    k8 = k4.reshape(S // 8, 2, D).mean(axis=1)
    kcat_ref[0] = jnp.concatenate(
        [k, k2, k4, k8, jnp.zeros((BLK, D), jnp.float32)], axis=0)
    v2 = v.reshape(S // 2, 2, D).mean(axis=1)
    v4 = v2.reshape(S // 4, 2, D).mean(axis=1)
    v8 = v4.reshape(S // 8, 2, D).mean(axis=1)
    vcat_ref[0] = jnp.concatenate(
        [v, v2, v4, v8, jnp.zeros((BLK, D), jnp.float32)], axis=0)


def _mask_sc_body(attn_hbm, val_hbm, band_hbm, forced_hbm, out_hbm,
                  band_v, forced_v, arow_v, ranks_v, valrow_v, outrow_v):
    nrows = attn_hbm.shape[0]
    wid = lax.axis_index("s") * 2 + lax.axis_index("c")
    per_w = nrows // 32
    base = wid * per_w

    pltpu.sync_copy(band_hbm, band_v)
    pltpu.sync_copy(forced_hbm, forced_v)

    idx = lax.iota(jnp.int32, NB)
    for t in range(per_w):
        row = base + t
        h = lax.shift_right_logical(row, 4)
        r = lax.bitwise_and(row, 15)
        pltpu.sync_copy(attn_hbm.at[row], arow_v)
        pltpu.sync_copy(val_hbm.at[h], valrow_v)
        a = arow_v[...]
        _, order = plsc.sort_key_val(a, idx, descending=True)
        plsc.store_scatter(ranks_v, [order], idx)
        ranks = ranks_v[...]
        band = plsc.load_gather(band_v, [jnp.full((NB,), r, jnp.int32),
                                         ranks])
        forced = forced_v[r]
        m = jnp.where(forced >= 0, forced, band)
        outrow_v[...] = jnp.minimum(m, valrow_v[...])
        pltpu.sync_copy(outrow_v, out_hbm.at[row])


def _mask_sc(attn2, val2):
    nrows = attn2.shape[0]
    kern = functools.partial(
        pl.kernel,
        mesh=plsc.VectorSubcoreMesh(core_axis_name="c", subcore_axis_name="s"),
        out_type=jax.ShapeDtypeStruct((nrows, NB), jnp.int32),
        compiler_params=pltpu.CompilerParams(needs_layout_passes=False),
        scratch_types=[
            pltpu.VMEM((NB, NB), jnp.int32),
            pltpu.VMEM((NB, NB), jnp.int32),
            pltpu.VMEM((NB,), jnp.float32),
            pltpu.VMEM((NB,), jnp.int32),
            pltpu.VMEM((NB,), jnp.int32),
            pltpu.VMEM((NB,), jnp.int32),
        ],
    )(_mask_sc_body)
    return kern(attn2, val2, jnp.asarray(_BAND_TABLE),
                jnp.asarray(_FORCED_TABLE))


def _attn_body(mask_ref, adj_ref, q_ref, kcat_ref, vcat_ref, o_ref):
    scale = np.float32(1.0 / math.sqrt(D))
    coli = lax.broadcasted_iota(jnp.int32, (BLK, BLK), 1)
    UNROLL = 4

    for i in range(NB):
        qs = q_ref[0, i * BLK:(i + 1) * BLK, :] * scale

        pd = mask_ref[0, i, i]
        idx = ((pd > 1).astype(jnp.int32) + (pd > 2).astype(jnp.int32)
               + (pd > 4).astype(jnp.int32))
        adj = adj_ref[idx]

        def logits(j, p):
            pe = jnp.maximum(p, 1)
            w = 128 // pe
            start = (4096 - 8192 // (2 * pe)) + j * w
            kblk = kcat_ref[0, pl.ds(start, BLK), :]
            vblk = vcat_ref[0, pl.ds(start, BLK), :]
            sj = jnp.dot(qs, kblk.T, precision=HI)
            offdiag = jnp.where((coli < w) & (p > 0), 0.0, NEG)
            sj = sj + jnp.where(j == i, adj, offdiag)
            return sj, vblk

        def body(t, carry):
            m, l, acc = carry
            js = [UNROLL * t + u for u in range(UNROLL)]
            ps = [mask_ref[0, i, js[0]]]
            ps += [jnp.where(js[u] <= i,
                             mask_ref[0, i, jnp.minimum(js[u], NB - 1)], 0)
                   for u in range(1, UNROLL)]
            sv = [logits(js[u], ps[u]) for u in range(UNROLL)]
            mx = jnp.max(sv[0][0], axis=1, keepdims=True)
            for u in range(1, UNROLL):
                mx = jnp.maximum(mx, jnp.max(sv[u][0], axis=1, keepdims=True))
            m_new = jnp.maximum(m, mx)
            alpha = jnp.exp(m - m_new)
            es = [jnp.exp(sj - m_new) for sj, _ in sv]
            l_new = l * alpha
            for e in es:
                l_new = l_new + jnp.sum(e, axis=1, keepdims=True)
            acc_new = acc * alpha
            for e, (_, vb) in zip(es, sv):
                acc_new = acc_new + jnp.dot(e, vb)
            return m_new, l_new, acc_new

        m0 = jnp.full((BLK, 1), NEG, jnp.float32)
        l0 = jnp.zeros((BLK, 1), jnp.float32)
        a0 = jnp.zeros((BLK, D), jnp.float32)
        m, l, acc = lax.fori_loop(0, (i + UNROLL) // UNROLL, body,
                                  (m0, l0, a0))
        o_ref[0, i * BLK:(i + 1) * BLK, :] = acc / l


def _estimate(q3, k3, interpret=False):
    H = q3.shape[0]
    return pl.pallas_call(
        _est_body,
        grid=(H,),
        in_specs=[pl.BlockSpec((1, S, D), lambda h: (h, 0, 0))] * 2,
        out_specs=[
            pl.BlockSpec((1, NB, NB), lambda h: (h, 0, 0)),
            pl.BlockSpec((1, 1, NB), lambda h: (h, 0, 0)),
        ],
        out_shape=[
            jax.ShapeDtypeStruct((H, NB, NB), jnp.float32),
            jax.ShapeDtypeStruct((H, 1, NB), jnp.int32),
        ],
        interpret=interpret,
    )(q3, k3)


def _pool(k3, v3, interpret=False):
    H = k3.shape[0]
    return pl.pallas_call(
        _pool_body,
        grid=(H,),
        in_specs=[pl.BlockSpec((1, S, D), lambda h: (h, 0, 0))] * 2,
        out_specs=[
            pl.BlockSpec((1, CAT, D), lambda h: (h, 0, 0)),
            pl.BlockSpec((1, CAT, D), lambda h: (h, 0, 0)),
        ],
        out_shape=[
            jax.ShapeDtypeStruct((H, CAT, D), jnp.float32),
            jax.ShapeDtypeStruct((H, CAT, D), jnp.float32),
        ],
        interpret=interpret,
    )(k3, v3)


def _attend(mask, q3, kcat, vcat, interpret=False):
    H = q3.shape[0]
    return pl.pallas_call(
        _attn_body,
        grid=(H,),
        in_specs=[
            pl.BlockSpec((1, NB, NB), lambda h: (h, 0, 0),
                         memory_space=pltpu.SMEM),
            pl.BlockSpec((4, BLK, BLK), lambda h: (0, 0, 0)),
            pl.BlockSpec((1, S, D), lambda h: (h, 0, 0)),
            pl.BlockSpec((1, CAT, D), lambda h: (h, 0, 0)),
            pl.BlockSpec((1, CAT, D), lambda h: (h, 0, 0)),
        ],
        out_specs=pl.BlockSpec((1, S, D), lambda h: (h, 0, 0)),
        out_shape=jax.ShapeDtypeStruct((H, S, D), jnp.float32),
        interpret=interpret,
    )(mask, _ADJ_TABLE, q3, kcat, vcat)


def kernel(q, k, v):
    B, H, s, d = q.shape
    assert s == S and d == D
    q3 = q.reshape(B * H, S, D)
    k3 = k.reshape(B * H, S, D)
    v3 = v.reshape(B * H, S, D)
    attn, val = _estimate(q3, k3)
    kcat, vcat = _pool(k3, v3)
    mask2 = _mask_sc(attn.reshape(B * H * NB, NB), val.reshape(B * H, NB))
    out = _attend(mask2.reshape(B * H, NB, NB), q3, kcat, vcat)
    return out.reshape(B, H, S, D)
